# baseline (speedup 1.0000x reference)
import functools

import jax
import jax.numpy as jnp
from jax import lax
from jax.experimental import pallas as pl
from jax.experimental.pallas import tpu as pltpu
from jax.experimental.pallas import tpu_sc as plsc

EN = 16
KN = 2
DM = 768
DF = 3072
TN = 2048
RB = 128
RP = TN * KN + EN * RB
NBLK = RP // RB

NC, NS = 2, 16
NW = NC * NS
TPW = TN // NW
VL = 16


def _gelu_exact(v):
    return 0.5 * v * (1.0 + jax.lax.erf(v * 0.7071067811865476))


def _gemm_body(be_ref, nbu_ref, x_ref, w0_ref, w1_ref, o_ref):
    j = pl.program_id(0)

    @pl.when(j < nbu_ref[0])
    def _():
        xb = x_ref[...].astype(jnp.bfloat16)
        w0b = w0_ref[0].astype(jnp.bfloat16)
        h = _gelu_exact(jnp.dot(xb, w0b, preferred_element_type=jnp.float32))
        o_ref[...] = jnp.dot(
            h.astype(jnp.bfloat16),
            w1_ref[0].astype(jnp.bfloat16),
            preferred_element_type=jnp.float32,
        )


def _grouped_gemm(block_expert, nbu, rep_x, w0, w1, interpret=False):
    return pl.pallas_call(
        _gemm_body,
        grid_spec=pltpu.PrefetchScalarGridSpec(
            num_scalar_prefetch=2,
            grid=(NBLK,),
            in_specs=[
                pl.BlockSpec((RB, DM), lambda j, be, nbu: (j, 0)),
                pl.BlockSpec((1, DM, DF), lambda j, be, nbu: (be[j], 0, 0)),
                pl.BlockSpec((1, DF, DM), lambda j, be, nbu: (be[j], 0, 0)),
            ],
            out_specs=pl.BlockSpec((RB, DM), lambda j, be, nbu: (j, 0)),
        ),
        out_shape=jax.ShapeDtypeStruct((RP, DM), jnp.float32),
        interpret=interpret,
    )(block_expert, nbu, rep_x, w0, w1)


@functools.lru_cache(maxsize=1)
def _sc_mesh():
    return plsc.VectorSubcoreMesh(
        core_axis_name="c", subcore_axis_name="s", num_cores=NC, num_subcores=NS
    )


def _sc_dispatch(x, pos_e, pos_o):
    return _sc_dispatch_kernel()(x, pos_e, pos_o)


@functools.lru_cache(maxsize=1)
def _sc_dispatch_kernel():
    return functools.partial(
        pl.kernel,
        out_type=jax.ShapeDtypeStruct((RP, DM), jnp.float32),
        mesh=_sc_mesh(),
        scratch_types=[
            pltpu.VMEM((TPW, DM), jnp.float32),
            pltpu.VMEM((TPW,), jnp.int32),
            pltpu.VMEM((TPW,), jnp.int32),
            pltpu.SemaphoreType.DMA,
            pltpu.SemaphoreType.DMA,
        ],
    )(_sc_dispatch_body)


def _sc_dispatch_body(x_hbm, pe_hbm, po_hbm, repx_hbm, xbuf, pe_v, po_v, sem0, sem1):
    wid = lax.axis_index("s") * NC + lax.axis_index("c")
    base = wid * TPW
    pltpu.sync_copy(x_hbm.at[pl.ds(base, TPW)], xbuf)
    pltpu.sync_copy(pe_hbm.at[pl.ds(base, TPW)], pe_v)
    pltpu.sync_copy(po_hbm.at[pl.ds(base, TPW)], po_v)
    c0 = pltpu.async_copy(xbuf, repx_hbm.at[pe_v], sem0)
    c1 = pltpu.async_copy(xbuf, repx_hbm.at[po_v], sem1)
    c0.wait()
    c1.wait()


def _sc_combine(y, pos_e, pos_o):
    return _sc_combine_kernel()(y, pos_e, pos_o)


@functools.lru_cache(maxsize=1)
def _sc_combine_kernel():
    return functools.partial(
        pl.kernel,
        out_type=jax.ShapeDtypeStruct((TN, DM), jnp.float32),
        mesh=_sc_mesh(),
        scratch_types=[
            pltpu.VMEM((TPW, DM), jnp.float32),
            pltpu.VMEM((TPW, DM), jnp.float32),
            pltpu.VMEM((TPW,), jnp.int32),
            pltpu.VMEM((TPW,), jnp.int32),
            pltpu.SemaphoreType.DMA,
            pltpu.SemaphoreType.DMA,
        ],
    )(_sc_combine_body)


def _sc_combine_body(y_hbm, pe_hbm, po_hbm, out_hbm, ge, go, pe_v, po_v, sem0, sem1):
    wid = lax.axis_index("s") * NC + lax.axis_index("c")
    base = wid * TPW
    pltpu.sync_copy(pe_hbm.at[pl.ds(base, TPW)], pe_v)
    pltpu.sync_copy(po_hbm.at[pl.ds(base, TPW)], po_v)
    c0 = pltpu.async_copy(y_hbm.at[pe_v], ge, sem0)
    c1 = pltpu.async_copy(y_hbm.at[po_v], go, sem1)
    c0.wait()
    c1.wait()

    def row_add(r, carry):
        for s in range(DM // VL):
            sl = pl.ds(s * VL, VL)
            ge[r, sl] = ge[r, sl] + go[r, sl]
        return carry

    lax.fori_loop(0, TPW, row_add, 0)
    pltpu.sync_copy(ge, out_hbm.at[pl.ds(base, TPW)])


def kernel(x, topk_index, w0, w1):
    e = topk_index.reshape(-1)
    oh = (e[:, None] == jnp.arange(EN, dtype=e.dtype)).astype(jnp.int32)
    cs = jnp.cumsum(oh, axis=0)
    rank = jnp.sum((cs - oh) * oh, axis=1)
    counts = cs[-1]
    padded = ((counts + RB - 1) // RB) * RB
    base = jnp.concatenate(
        [jnp.zeros((1,), jnp.int32), jnp.cumsum(padded)[:-1].astype(jnp.int32)]
    )
    pos = rank + jnp.sum(oh * base[None, :], axis=1)
    blk_base = base // RB
    jidx = jnp.arange(NBLK, dtype=jnp.int32)
    block_expert = (
        jnp.sum((blk_base[None, :] <= jidx[:, None]).astype(jnp.int32), axis=1) - 1
    ).astype(jnp.int32)
    pos2 = pos.reshape(TN, KN)
    pos_e = pos2[:, 0]
    pos_o = pos2[:, 1]
    nbu = (jnp.sum(padded) // RB).astype(jnp.int32).reshape(1)
    rep_x = _sc_dispatch(x, pos_e, pos_o)
    y = _grouped_gemm(block_expert, nbu, rep_x, w0, w1)
    return _sc_combine(y, pos_e, pos_o)


# speedup vs baseline: 1.1203x; 1.1203x over previous
import functools

import jax
import jax.numpy as jnp
from jax import lax
from jax.experimental import pallas as pl
from jax.experimental.pallas import tpu as pltpu
from jax.experimental.pallas import tpu_sc as plsc

EN = 16
KN = 2
DM = 768
DF = 3072
TN = 2048
RB = 256
RP = TN * KN + EN * RB
NBLK = RP // RB

NC, NS = 2, 16
NW = NC * NS
TPW = TN // NW
VL = 16


def _gelu_exact(v):
    return 0.5 * v * (1.0 + jax.lax.erf(v * 0.7071067811865476))


def _gemm_body(be_ref, nbu_ref, x_ref, w0_ref, w1_ref, o_ref):
    j = pl.program_id(0)

    @pl.when(j < nbu_ref[0])
    def _():
        xb = x_ref[...].astype(jnp.bfloat16)
        w0b = w0_ref[0].astype(jnp.bfloat16)
        h = _gelu_exact(jnp.dot(xb, w0b, preferred_element_type=jnp.float32))
        o_ref[...] = jnp.dot(
            h.astype(jnp.bfloat16),
            w1_ref[0].astype(jnp.bfloat16),
            preferred_element_type=jnp.float32,
        )


def _grouped_gemm(block_expert, nbu, rep_x, w0, w1, interpret=False):
    return pl.pallas_call(
        _gemm_body,
        grid_spec=pltpu.PrefetchScalarGridSpec(
            num_scalar_prefetch=2,
            grid=(NBLK,),
            in_specs=[
                pl.BlockSpec(
                    (RB, DM),
                    lambda j, be, nbu: (jnp.minimum(j, nbu[0] - 1), 0),
                ),
                pl.BlockSpec(
                    (1, DM, DF),
                    lambda j, be, nbu: (be[jnp.minimum(j, nbu[0] - 1)], 0, 0),
                ),
                pl.BlockSpec(
                    (1, DF, DM),
                    lambda j, be, nbu: (be[jnp.minimum(j, nbu[0] - 1)], 0, 0),
                ),
            ],
            out_specs=pl.BlockSpec((RB, DM), lambda j, be, nbu: (j, 0)),
        ),
        out_shape=jax.ShapeDtypeStruct((RP, DM), jnp.float32),
        interpret=interpret,
    )(block_expert, nbu, rep_x, w0, w1)


@functools.lru_cache(maxsize=1)
def _sc_mesh():
    return plsc.VectorSubcoreMesh(
        core_axis_name="c", subcore_axis_name="s", num_cores=NC, num_subcores=NS
    )


def _sc_dispatch(x, pos_e, pos_o):
    return _sc_dispatch_kernel()(x, pos_e, pos_o)


@functools.lru_cache(maxsize=1)
def _sc_dispatch_kernel():
    return functools.partial(
        pl.kernel,
        out_type=jax.ShapeDtypeStruct((RP, DM), jnp.float32),
        mesh=_sc_mesh(),
        scratch_types=[
            pltpu.VMEM((TPW, DM), jnp.float32),
            pltpu.VMEM((TPW,), jnp.int32),
            pltpu.VMEM((TPW,), jnp.int32),
            pltpu.SemaphoreType.DMA,
            pltpu.SemaphoreType.DMA,
        ],
    )(_sc_dispatch_body)


def _sc_dispatch_body(x_hbm, pe_hbm, po_hbm, repx_hbm, xbuf, pe_v, po_v, sem0, sem1):
    wid = lax.axis_index("s") * NC + lax.axis_index("c")
    base = wid * TPW
    pltpu.sync_copy(x_hbm.at[pl.ds(base, TPW)], xbuf)
    pltpu.sync_copy(pe_hbm.at[pl.ds(base, TPW)], pe_v)
    pltpu.sync_copy(po_hbm.at[pl.ds(base, TPW)], po_v)
    c0 = pltpu.async_copy(xbuf, repx_hbm.at[pe_v], sem0)
    c1 = pltpu.async_copy(xbuf, repx_hbm.at[po_v], sem1)
    c0.wait()
    c1.wait()


def _sc_combine(y, pos_e, pos_o):
    return _sc_combine_kernel()(y, pos_e, pos_o)


@functools.lru_cache(maxsize=1)
def _sc_combine_kernel():
    return functools.partial(
        pl.kernel,
        out_type=jax.ShapeDtypeStruct((TN, DM), jnp.float32),
        mesh=_sc_mesh(),
        scratch_types=[
            pltpu.VMEM((TPW, DM), jnp.float32),
            pltpu.VMEM((TPW, DM), jnp.float32),
            pltpu.VMEM((TPW,), jnp.int32),
            pltpu.VMEM((TPW,), jnp.int32),
            pltpu.SemaphoreType.DMA,
            pltpu.SemaphoreType.DMA,
        ],
    )(_sc_combine_body)


def _sc_combine_body(y_hbm, pe_hbm, po_hbm, out_hbm, ge, go, pe_v, po_v, sem0, sem1):
    wid = lax.axis_index("s") * NC + lax.axis_index("c")
    base = wid * TPW
    pltpu.sync_copy(pe_hbm.at[pl.ds(base, TPW)], pe_v)
    pltpu.sync_copy(po_hbm.at[pl.ds(base, TPW)], po_v)
    c0 = pltpu.async_copy(y_hbm.at[pe_v], ge, sem0)
    c1 = pltpu.async_copy(y_hbm.at[po_v], go, sem1)
    c0.wait()
    c1.wait()

    def row_add(r, carry):
        for s in range(DM // VL):
            sl = pl.ds(s * VL, VL)
            ge[r, sl] = ge[r, sl] + go[r, sl]
        return carry

    lax.fori_loop(0, TPW, row_add, 0)
    pltpu.sync_copy(ge, out_hbm.at[pl.ds(base, TPW)])


def kernel(x, topk_index, w0, w1):
    e = topk_index.reshape(-1)
    oh = (e[:, None] == jnp.arange(EN, dtype=e.dtype)).astype(jnp.int32)
    cs = jnp.cumsum(oh, axis=0)
    rank = jnp.sum((cs - oh) * oh, axis=1)
    counts = cs[-1]
    padded = ((counts + RB - 1) // RB) * RB
    base = jnp.concatenate(
        [jnp.zeros((1,), jnp.int32), jnp.cumsum(padded)[:-1].astype(jnp.int32)]
    )
    pos = rank + jnp.sum(oh * base[None, :], axis=1)
    blk_base = base // RB
    jidx = jnp.arange(NBLK, dtype=jnp.int32)
    block_expert = (
        jnp.sum((blk_base[None, :] <= jidx[:, None]).astype(jnp.int32), axis=1) - 1
    ).astype(jnp.int32)
    pos2 = pos.reshape(TN, KN)
    pos_e = pos2[:, 0]
    pos_o = pos2[:, 1]
    nbu = (jnp.sum(padded) // RB).astype(jnp.int32).reshape(1)
    rep_x = _sc_dispatch(x, pos_e, pos_o)
    y = _grouped_gemm(block_expert, nbu, rep_x, w0, w1)
    return _sc_combine(y, pos_e, pos_o)
